# baseline (speedup 1.0000x reference)
import functools

import jax
import jax.numpy as jnp
from jax import lax
from jax.experimental import pallas as pl
from jax.experimental.pallas import tpu as pltpu
from jax.experimental.pallas import tpu_sc as plsc

_RADII = [0.2, 0.4]
_MAX_POINTS = [32, 64]

_N = 8192
_NB = 8
_L = 16
_NW = 32
_QPW = _N // _NW
_GPW = _QPW // _L

_R2_0 = _RADII[0] * _RADII[0]
_R2_1 = _RADII[1] * _RADII[1]


def _sc_body(kx_hbm, ky_hbm, kz_hbm, kb_hbm, glo_hbm, ghi_hbm,
             out0_hbm, out1_hbm,
             kxv, kyv, kzv, kbv, glov, ghiv, c0v, c1v):
    wid = lax.axis_index("s") * 2 + lax.axis_index("c")
    qbase = wid * _QPW

    pltpu.sync_copy(kx_hbm, kxv)
    pltpu.sync_copy(ky_hbm, kyv)
    pltpu.sync_copy(kz_hbm, kzv)
    pltpu.sync_copy(kb_hbm, kbv)
    pltpu.sync_copy(glo_hbm.at[pl.ds(wid * _GPW, _GPW)], glov)
    pltpu.sync_copy(ghi_hbm.at[pl.ds(wid * _GPW, _GPW)], ghiv)

    glo_all = glov[...]
    ghi_all = ghiv[...]

    for g in range(_GPW):
        qoff = qbase + g * _L
        qx = kxv[pl.ds(qoff, _L)]
        qy = kyv[pl.ds(qoff, _L)]
        qz = kzv[pl.ds(qoff, _L)]
        qb = kbv[pl.ds(qoff, _L)]

        lo = glo_all[g]
        hi = ghi_all[g]
        trip = (hi - lo + _L - 1) // _L

        def body(j, carry, lo=lo, qx=qx, qy=qy, qz=qz, qb=qb):
            c0, c1 = carry
            koff = lo + j * _L
            kxg = kxv[pl.ds(koff, _L)]
            kyg = kyv[pl.ds(koff, _L)]
            kzg = kzv[pl.ds(koff, _L)]
            kbg = kbv[pl.ds(koff, _L)]
            for l in range(_L):
                dx = kxg[l] - qx
                dy = kyg[l] - qy
                dz = kzg[l] - qz
                d2 = dx * dx + dy * dy + dz * dz
                same = kbg[l] == qb
                c0 = c0 + jnp.where((d2 <= _R2_0) & same, 1.0, 0.0)
                c1 = c1 + jnp.where((d2 <= _R2_1) & same, 1.0, 0.0)
            return c0, c1

        zero = jnp.zeros((_L,), jnp.float32)
        c0, c1 = lax.fori_loop(0, trip, body, (zero, zero))
        c0v[pl.ds(g * _L, _L)] = c0
        c1v[pl.ds(g * _L, _L)] = c1

    pltpu.sync_copy(c0v, out0_hbm.at[pl.ds(qbase, _QPW)])
    pltpu.sync_copy(c1v, out1_hbm.at[pl.ds(qbase, _QPW)])


_sc_counts = pl.kernel(
    _sc_body,
    out_type=(
        jax.ShapeDtypeStruct((_N,), jnp.float32),
        jax.ShapeDtypeStruct((_N,), jnp.float32),
    ),
    mesh=plsc.VectorSubcoreMesh(core_axis_name="c", subcore_axis_name="s"),
    scratch_types=[
        pltpu.VMEM((_N,), jnp.float32),
        pltpu.VMEM((_N,), jnp.float32),
        pltpu.VMEM((_N,), jnp.float32),
        pltpu.VMEM((_N,), jnp.int32),
        pltpu.VMEM((_GPW,), jnp.int32),
        pltpu.VMEM((_GPW,), jnp.int32),
        pltpu.VMEM((_QPW,), jnp.float32),
        pltpu.VMEM((_QPW,), jnp.float32),
    ],
)


@jax.jit
def kernel(x, pos, batch):
    batch = batch.astype(jnp.int32)
    kx = pos[:, 0]
    ky = pos[:, 1]
    kz = pos[:, 2]

    bids = jnp.arange(_NB, dtype=batch.dtype)
    seg_start = jnp.searchsorted(batch, bids, side="left").astype(jnp.int32)
    seg_end = jnp.searchsorted(batch, bids, side="right").astype(jnp.int32)
    b2 = batch.reshape(_N // _L, _L)
    glo = (seg_start[b2[:, 0]] // _L) * _L
    ghi = seg_end[b2[:, -1]]

    c0, c1 = _sc_counts(kx, ky, kz, batch, glo, ghi)

    cnt0 = jnp.minimum(c0, float(_MAX_POINTS[0])) / float(_MAX_POINTS[0])
    cnt1 = jnp.minimum(c1, float(_MAX_POINTS[1])) / float(_MAX_POINTS[1])
    feats = jnp.concatenate([x, pos, cnt0[:, None], cnt1[:, None]], axis=1)
    return (feats, pos, batch)
